# baseline (speedup 1.0000x reference)
import functools

import jax
import jax.numpy as jnp
from jax import lax
from jax.experimental import pallas as pl
from jax.experimental.pallas import tpu as pltpu
from jax.experimental.pallas import tpu_sc as plsc

_EPS = 1e-3
_NC = 2
_NS = 16
_NW = _NC * _NS


def _sc_gather_slice(table, idx_flat, k, nm_sl, chunk):
    d = table.shape[1]
    per_w = nm_sl // _NW
    n_ch = per_w // chunk
    assert per_w % chunk == 0 and chunk % 8 == 0 and per_w % 8 == 0

    mesh = plsc.VectorSubcoreMesh(core_axis_name="c", subcore_axis_name="s")

    @functools.partial(
        pl.kernel,
        mesh=mesh,
        out_type=jax.ShapeDtypeStruct((nm_sl, d), jnp.float32),
        scratch_types=[
            pltpu.VMEM((per_w,), jnp.int32),
            pltpu.VMEM((chunk, d), jnp.float32),
            pltpu.VMEM((chunk, d), jnp.float32),
            pltpu.SemaphoreType.DMA,
            pltpu.SemaphoreType.DMA,
        ],
    )
    def gather_kernel(table_hbm, idx_hbm, out_hbm, idx_v, rows0, rows1, sem0, sem1):
        wid = lax.axis_index("s") * _NC + lax.axis_index("c")
        out_base = wid * per_w
        idx_base = k * nm_sl + out_base
        pltpu.sync_copy(idx_hbm.at[pl.ds(idx_base, per_w)], idx_v)
        bufs = (rows0, rows1)
        sems = (sem0, sem1)

        def start(c):
            return pltpu.async_copy(
                table_hbm.at[idx_v.at[pl.ds(c * chunk, chunk)]],
                bufs[c % 2], sems[c % 2])

        def scatter(c):
            pltpu.sync_copy(bufs[c % 2],
                            out_hbm.at[pl.ds(out_base + c * chunk, chunk)])

        h_prev = start(0)
        for c in range(1, n_ch):
            h = start(c)
            h_prev.wait()
            scatter(c - 1)
            h_prev = h
        h_prev.wait()
        scatter(n_ch - 1)

    return gather_kernel(table, idx_flat)


def _softplus_fast(x):
    return jnp.log1p(jnp.exp(x))


def _softplus(x):
    return jnp.maximum(x, 0.0) + jnp.log1p(jnp.exp(-jnp.abs(x)))


def _tc_body(m, a_len, atom_ref, g_ref, nb_ref, ws_ref, wn_ref,
             b_ref, s2_ref, b2_ref, out_ref):
    a = atom_ref[...]
    s = jnp.dot(a, ws_ref[...], preferred_element_type=jnp.float32)
    s = s + b_ref[...]
    nb = nb_ref[...]
    nb = nb.reshape(nb.shape[0] * m, nb.shape[2])
    xin = jnp.concatenate([g_ref[...], nb], axis=1)
    x = jnp.dot(xin, wn_ref[...], preferred_element_type=jnp.float32)
    bsz = a.shape[0]
    x = x.reshape(bsz, m, 2 * a_len) + s[:, None, :]
    filt = 0.5 * jnp.tanh(0.5 * x[:, :, :a_len]) + 0.5
    core = _softplus_fast(x[:, :, a_len:])
    red = jnp.sum(filt * core, axis=1)
    red = red * s2_ref[...] + b2_ref[...]
    out_ref[...] = _softplus(a + red)


def _tc_slice(atom, g_sl, nb3, ws, wn, bvec, s2, b2,
              k, n_sl, block):
    n, a_len = atom.shape
    nm_sl = g_sl.shape[0]
    m = nm_sl // n_sl
    e_len = nb3.shape[2]
    assert n_sl % block == 0
    grid = (n_sl // block,)
    blk0 = k * (n_sl // block)
    body = functools.partial(_tc_body, m, a_len)
    return pl.pallas_call(
        body,
        grid=grid,
        in_specs=[
            pl.BlockSpec((block, a_len), lambda i: (blk0 + i, 0)),
            pl.BlockSpec((block * m, a_len), lambda i: (i, 0)),
            pl.BlockSpec((block, m, e_len), lambda i: (blk0 + i, 0, 0)),
            pl.BlockSpec((a_len, 2 * a_len), lambda i: (0, 0)),
            pl.BlockSpec((a_len + e_len, 2 * a_len), lambda i: (0, 0)),
            pl.BlockSpec((1, 2 * a_len), lambda i: (0, 0)),
            pl.BlockSpec((1, a_len), lambda i: (0, 0)),
            pl.BlockSpec((1, a_len), lambda i: (0, 0)),
        ],
        out_specs=pl.BlockSpec((block, a_len), lambda i: (i, 0)),
        out_shape=jax.ShapeDtypeStruct((n_sl, a_len), jnp.float32),
        compiler_params=pltpu.CompilerParams(
            dimension_semantics=("arbitrary",),
        ),
    )(atom, g_sl, nb3, ws, wn, bvec, s2, b2)


def kernel(atom_in_fea, nbr_fea, nbr_fea_idx, W_fc, b_fc,
           bn1_gamma, bn1_beta, bn1_mean, bn1_var,
           bn2_gamma, bn2_beta, bn2_mean, bn2_var):
    n, m = nbr_fea_idx.shape
    a_len = atom_in_fea.shape[1]

    scale1 = bn1_gamma * lax.rsqrt(bn1_var + _EPS)
    wp = W_fc * scale1[None, :]
    bp = b_fc * scale1 + (bn1_beta - bn1_mean * scale1)
    ws = wp[:a_len]
    wn = wp[a_len:]
    scale2 = bn2_gamma * lax.rsqrt(bn2_var + _EPS)
    bias2 = bn2_beta - bn2_mean * scale2

    idx_flat = nbr_fea_idx.reshape(-1).astype(jnp.int32)
    bvec = bp.reshape(1, -1)
    s2 = scale2.reshape(1, -1)
    b2 = bias2.reshape(1, -1)

    n_slices = 5
    n_sl = n // n_slices
    nm_sl = n_sl * m
    outs = []
    for k in range(n_slices):
        g_k = _sc_gather_slice(atom_in_fea, idx_flat, k, nm_sl, chunk=200)
        outs.append(_tc_slice(atom_in_fea, g_k, nbr_fea, ws, wn,
                              bvec, s2, b2, k, n_sl, block=200))
    return jnp.concatenate(outs, axis=0)


# speedup vs baseline: 1.1489x; 1.1489x over previous
import functools

import jax
import jax.numpy as jnp
from jax import lax
from jax.experimental import pallas as pl
from jax.experimental.pallas import tpu as pltpu
from jax.experimental.pallas import tpu_sc as plsc

_EPS = 1e-3
_NC = 2
_NS = 16
_NW = _NC * _NS


def _sc_gather_slice(table, idx_flat, edge0, nm_sl, chunk):
    d = table.shape[1]
    per_w = nm_sl // _NW
    n_ch = per_w // chunk
    assert per_w % chunk == 0 and chunk % 8 == 0 and per_w % 8 == 0

    mesh = plsc.VectorSubcoreMesh(core_axis_name="c", subcore_axis_name="s")

    @functools.partial(
        pl.kernel,
        mesh=mesh,
        out_type=jax.ShapeDtypeStruct((nm_sl, d), jnp.float32),
        scratch_types=[
            pltpu.VMEM((per_w,), jnp.int32),
            pltpu.VMEM((chunk, d), jnp.float32),
            pltpu.VMEM((chunk, d), jnp.float32),
            pltpu.SemaphoreType.DMA,
            pltpu.SemaphoreType.DMA,
        ],
    )
    def gather_kernel(table_hbm, idx_hbm, out_hbm, idx_v, rows0, rows1, sem0, sem1):
        wid = lax.axis_index("s") * _NC + lax.axis_index("c")
        out_base = wid * per_w
        idx_base = edge0 + out_base
        pltpu.sync_copy(idx_hbm.at[pl.ds(idx_base, per_w)], idx_v)
        bufs = (rows0, rows1)
        sems = (sem0, sem1)

        def start(c):
            return pltpu.async_copy(
                table_hbm.at[idx_v.at[pl.ds(c * chunk, chunk)]],
                bufs[c % 2], sems[c % 2])

        def scatter(c):
            pltpu.sync_copy(bufs[c % 2],
                            out_hbm.at[pl.ds(out_base + c * chunk, chunk)])

        h_prev = start(0)
        for c in range(1, n_ch):
            h = start(c)
            h_prev.wait()
            scatter(c - 1)
            h_prev = h
        h_prev.wait()
        scatter(n_ch - 1)

    return gather_kernel(table, idx_flat)


def _softplus_fast(x):
    return jnp.log1p(jnp.exp(x))


def _softplus(x):
    return jnp.maximum(x, 0.0) + jnp.log1p(jnp.exp(-jnp.abs(x)))


def _tc_body(m, a_len, atom_ref, g_ref, nb_ref, ws_ref, wn_ref,
             b_ref, s2_ref, b2_ref, out_ref):
    a = atom_ref[...]
    s = jnp.dot(a, ws_ref[...], preferred_element_type=jnp.float32)
    s = s + b_ref[...]
    xin = jnp.concatenate([g_ref[...], nb_ref[...]], axis=1)
    x = jnp.dot(xin, wn_ref[...], preferred_element_type=jnp.float32)
    bsz = a.shape[0]
    x = x.reshape(bsz, m, 2 * a_len) + s[:, None, :]
    t = jnp.tanh(x[:, :, :a_len])
    core = _softplus_fast(x[:, :, a_len:])
    red = jnp.sum((t + 1.0) * core, axis=1)
    red = red * s2_ref[...] + b2_ref[...]
    out_ref[...] = _softplus(a + red)


def _tc_slice(atom, g_sl, nb_flat, ws, wn, bvec, s2, b2,
              node0, n_sl, block):
    n, a_len = atom.shape
    nm_sl = g_sl.shape[0]
    m = nm_sl // n_sl
    e_len = nb_flat.shape[1]
    assert n_sl % block == 0 and node0 % block == 0
    grid = (n_sl // block,)
    blk0 = node0 // block
    body = functools.partial(_tc_body, m, a_len)
    return pl.pallas_call(
        body,
        grid=grid,
        in_specs=[
            pl.BlockSpec((block, a_len), lambda i: (blk0 + i, 0)),
            pl.BlockSpec((block * m, a_len), lambda i: (i, 0)),
            pl.BlockSpec((block * m, e_len), lambda i: (blk0 + i, 0)),
            pl.BlockSpec((a_len, 2 * a_len), lambda i: (0, 0)),
            pl.BlockSpec((a_len + e_len, 2 * a_len), lambda i: (0, 0)),
            pl.BlockSpec((1, 2 * a_len), lambda i: (0, 0)),
            pl.BlockSpec((1, a_len), lambda i: (0, 0)),
            pl.BlockSpec((1, a_len), lambda i: (0, 0)),
        ],
        out_specs=pl.BlockSpec((block, a_len), lambda i: (i, 0)),
        out_shape=jax.ShapeDtypeStruct((n_sl, a_len), jnp.float32),
        compiler_params=pltpu.CompilerParams(
            dimension_semantics=("arbitrary",),
        ),
    )(atom, g_sl, nb_flat, ws, wn, bvec, s2, b2)


def kernel(atom_in_fea, nbr_fea, nbr_fea_idx, W_fc, b_fc,
           bn1_gamma, bn1_beta, bn1_mean, bn1_var,
           bn2_gamma, bn2_beta, bn2_mean, bn2_var):
    n, m = nbr_fea_idx.shape
    a_len = atom_in_fea.shape[1]

    scale1 = bn1_gamma * lax.rsqrt(bn1_var + _EPS)
    half = jnp.concatenate([jnp.full((a_len,), 0.5, jnp.float32),
                            jnp.ones((a_len,), jnp.float32)])
    scale1h = scale1 * half
    wp = W_fc * scale1h[None, :]
    bp = b_fc * scale1h + (bn1_beta - bn1_mean * scale1) * half
    ws = wp[:a_len]
    wn = wp[a_len:]
    scale2 = bn2_gamma * lax.rsqrt(bn2_var + _EPS)
    bias2 = bn2_beta - bn2_mean * scale2

    idx_flat = nbr_fea_idx.reshape(-1).astype(jnp.int32)
    nb_flat = nbr_fea.reshape(n * m, -1)
    bvec = bp.reshape(1, -1)
    s2 = (0.5 * scale2).reshape(1, -1)
    b2 = bias2.reshape(1, -1)

    slices = [(0, 400), (400, 1600), (2000, 2000), (4000, 2000),
              (6000, 2000), (8000, 2000)]
    outs = []
    for node0, n_sl in slices:
        g_k = _sc_gather_slice(atom_in_fea, idx_flat, node0 * m, n_sl * m,
                               chunk=400)
        outs.append(_tc_slice(atom_in_fea, g_k, nb_flat, ws, wn,
                              bvec, s2, b2, node0, n_sl, block=200))
    return jnp.concatenate(outs, axis=0)
